# Initial kernel scaffold; baseline (speedup 1.0000x reference)
import functools

import jax
import jax.numpy as jnp
from jax import lax
from jax.experimental import pallas as pl
from jax.experimental.pallas import tpu as pltpu
from jax.experimental.pallas import tpu_sc as plsc

N = 50000
D = 32
E = 800000
NC = 2
NS = 16
NW = NC * NS
C = 128
NCHUNK = -(-E // (NW * C))
E_PAD = NW * NCHUNK * C
N_PAD = 50176
TRASH = N_PAD - N
RPT = N_PAD // NS
R_BLK = 3584
N_BLKS = N_PAD // R_BLK
NLIST = 7

SRC_IS_A = (False, False, True, True, True, False, False)


def _seg_mean_sc(feat_a, feat_b, dst_all, src_all, zeros2d, zeros1d):
    mesh = plsc.VectorSubcoreMesh(
        core_axis_name="c", subcore_axis_name="s", num_cores=NC,
        num_subcores=NS)

    @functools.partial(
        pl.kernel,
        out_type=(
            jax.ShapeDtypeStruct((NLIST, NC, N_PAD, D), jnp.float32),
            jax.ShapeDtypeStruct((NLIST, NC, N_PAD), jnp.float32),
        ),
        mesh=mesh,
        scratch_types=[
            pltpu.VMEM_SHARED((N_PAD, D), jnp.float32),
            pltpu.VMEM_SHARED((N_PAD,), jnp.float32),
            pltpu.VMEM((NCHUNK, C), jnp.int32),
            pltpu.VMEM((NCHUNK, C), jnp.int32),
            pltpu.VMEM((C, D), jnp.float32),
            pltpu.VMEM((C, D), jnp.float32),
            pltpu.VMEM((C,), jnp.float32),
            pltpu.SemaphoreType.DMA,
            pltpu.SemaphoreType.DMA,
        ],
    )
    def seg_kernel(feat_a_hbm, feat_b_hbm, dst_hbm, src_hbm, z2_hbm, z1_hbm,
                   acc_out, deg_out, acc_sh, deg_sh, dst_buf, src_buf,
                   rows0, rows1, ones_b, sem0, sem1):
        cid = lax.axis_index("c")
        sid = lax.axis_index("s")
        wid = cid * NS + sid

        for i in range(C // 16):
            ones_b[pl.ds(16 * i, 16)] = jnp.ones((16,), jnp.float32)

        for l in range(NLIST):
            feat_hbm = feat_a_hbm if SRC_IS_A[l] else feat_b_hbm

            pltpu.sync_copy(z2_hbm, acc_sh.at[pl.ds(sid * RPT, RPT)])
            pltpu.sync_copy(z1_hbm, deg_sh.at[pl.ds(sid * RPT, RPT)])

            pltpu.sync_copy(dst_hbm.at[l, pl.ds(wid * NCHUNK, NCHUNK)],
                            dst_buf)
            pltpu.sync_copy(src_hbm.at[l, pl.ds(wid * NCHUNK, NCHUNK)],
                            src_buf)
            plsc.subcore_barrier()

            def gstart(c, buf, sem):
                pltpu.async_copy(feat_hbm.at[src_buf.at[c]], buf, sem)

            def gwait(c, buf, sem):
                pltpu.make_async_copy(feat_hbm.at[src_buf.at[c]], buf,
                                      sem).wait()

            def scat(c, buf):
                pltpu.sync_copy(buf, acc_sh.at[dst_buf.at[c]], add=True)
                pltpu.sync_copy(ones_b, deg_sh.at[dst_buf.at[c]], add=True)

            gstart(0, rows0, sem0)

            def pair(i, carry):
                c0 = 2 * i
                gstart(c0 + 1, rows1, sem1)
                gwait(c0, rows0, sem0)
                scat(c0, rows0)
                gstart(c0 + 2, rows0, sem0)
                gwait(c0 + 1, rows1, sem1)
                scat(c0 + 1, rows1)
                return carry

            lax.fori_loop(0, NCHUNK // 2 - 1, pair, 0)
            c0 = NCHUNK - 2
            gstart(c0 + 1, rows1, sem1)
            gwait(c0, rows0, sem0)
            scat(c0, rows0)
            gwait(c0 + 1, rows1, sem1)
            scat(c0 + 1, rows1)

            plsc.subcore_barrier()

            pltpu.sync_copy(acc_sh.at[pl.ds(sid * RPT, RPT)],
                            acc_out.at[l, cid, pl.ds(sid * RPT, RPT)])
            pltpu.sync_copy(deg_sh.at[pl.ds(sid * RPT, RPT)],
                            deg_out.at[l, cid, pl.ds(sid * RPT, RPT)])
            plsc.subcore_barrier()

    return seg_kernel(feat_a, feat_b, dst_all, src_all, zeros2d, zeros1d)


def _update_tc(feat, accs, degs, slot_map, Ws, bWs, U1_w, U1_b, prelu_a,
               U2_w, U2_b):
    n_agg = len(accs)

    def body(feat_ref, *refs):
        a_refs = refs[:n_agg]
        d_refs = refs[n_agg:2 * n_agg]
        w_refs = refs[2 * n_agg:2 * n_agg + 4]
        bw_refs = refs[2 * n_agg + 4:2 * n_agg + 8]
        u1w_ref, u1b_ref, pa_ref, u2w_ref, u2b_ref, out_ref = \
            refs[2 * n_agg + 8:]

        u1 = u1w_ref[...]
        x = feat_ref[...]
        h = jnp.dot(x, u1[0:D, :], preferred_element_type=jnp.float32)
        h = h + u1b_ref[...]

        means = {}
        for slot in range(4):
            ai = slot_map[slot]
            if ai not in means:
                acc = a_refs[ai][...]
                dg = d_refs[ai][...]
                dsum = jnp.maximum(dg[0] + dg[1], 1.0)
                means[ai] = (acc[0] + acc[1]) * (1.0 / dsum)[:, None]
            s = means[ai]
            u1_blk = u1[D * (slot + 1):D * (slot + 2), :]
            g = jnp.dot(w_refs[slot][...], u1_blk,
                        preferred_element_type=jnp.float32)
            h = h + jnp.dot(s, g, preferred_element_type=jnp.float32)
            h = h + jnp.dot(bw_refs[slot][...], u1_blk,
                            preferred_element_type=jnp.float32)

        a = pa_ref[0, 0]
        h = jnp.where(h >= 0, h, a * h)
        y = jnp.dot(h, u2w_ref[...], preferred_element_type=jnp.float32)
        out_ref[...] = y + u2b_ref[...]

    in_specs = (
        [pl.BlockSpec((R_BLK, D), lambda i: (i, 0))]
        + [pl.BlockSpec((NC, R_BLK, D), lambda i: (0, i, 0))] * n_agg
        + [pl.BlockSpec((NC, R_BLK), lambda i: (0, i))] * n_agg
        + [pl.BlockSpec((D, D), lambda i: (0, 0))] * 4
        + [pl.BlockSpec((1, D), lambda i: (0, 0))] * 4
        + [pl.BlockSpec((5 * D, 2 * D), lambda i: (0, 0)),
           pl.BlockSpec((1, 2 * D), lambda i: (0, 0)),
           pl.BlockSpec((1, 1), lambda i: (0, 0)),
           pl.BlockSpec((2 * D, D), lambda i: (0, 0)),
           pl.BlockSpec((1, D), lambda i: (0, 0))]
    )
    return pl.pallas_call(
        body,
        grid=(N_BLKS,),
        in_specs=in_specs,
        out_specs=pl.BlockSpec((R_BLK, D), lambda i: (i, 0)),
        out_shape=jax.ShapeDtypeStruct((N_PAD, D), jnp.float32),
    )(feat, *accs, *degs, *Ws,
      *[b.reshape(1, D) for b in bWs], U1_w, U1_b.reshape(1, 2 * D),
      prelu_a.reshape(1, 1), U2_w, U2_b.reshape(1, D))


def _prep_edges(edges):
    pad = E_PAD - E
    dst = jnp.concatenate(
        [edges[0], N + (jnp.arange(pad, dtype=jnp.int32) % TRASH)])
    src = jnp.concatenate([edges[1], jnp.zeros((pad,), jnp.int32)])
    return dst.reshape(NW * NCHUNK, C), src.reshape(NW * NCHUNK, C)


def kernel(feature_a, feature_b, edges_ab_pos, edges_ab_neg, edges_ba_pos,
           edges_ba_neg, edges_aa_pos, edges_aa_neg, edges_bb_pos,
           edges_bb_neg, W0, bW0, W1, bW1, W2, bW2, W3, bW3, W4, bW4, W5,
           bW5, W6, bW6, W7, bW7, U1_w, U1_b, prelu_a, U2_w, U2_b):
    feat_a = jnp.pad(feature_a, ((0, N_PAD - N), (0, 0)))
    feat_b = jnp.pad(feature_b, ((0, N_PAD - N), (0, 0)))

    lists = (edges_ab_pos, edges_ab_neg, edges_aa_pos, edges_aa_neg,
             edges_ba_pos, edges_bb_pos, edges_bb_neg)
    prepped = [_prep_edges(e) for e in lists]
    dst_all = jnp.stack([p[0] for p in prepped])
    src_all = jnp.stack([p[1] for p in prepped])

    zeros2d = jnp.zeros((RPT, D), jnp.float32)
    zeros1d = jnp.zeros((RPT,), jnp.float32)

    acc, deg = _seg_mean_sc(feat_a, feat_b, dst_all, src_all, zeros2d,
                            zeros1d)

    new_a = _update_tc(
        feat_a, [acc[0], acc[1], acc[2], acc[3]],
        [deg[0], deg[1], deg[2], deg[3]], (0, 1, 2, 3),
        (W0, W1, W2, W3), (bW0, bW1, bW2, bW3), U1_w, U1_b, prelu_a,
        U2_w, U2_b)
    new_b = _update_tc(
        feat_b, [acc[4], acc[5], acc[6]],
        [deg[4], deg[5], deg[6]], (0, 0, 1, 2),
        (W4, W5, W6, W7), (bW4, bW5, bW6, bW7), U1_w, U1_b, prelu_a,
        U2_w, U2_b)
    return new_a[:N], new_b[:N]


# speedup vs baseline: 10.9983x; 10.9983x over previous
import functools

import jax
import jax.numpy as jnp
from jax import lax
from jax.experimental import pallas as pl
from jax.experimental.pallas import tpu as pltpu
from jax.experimental.pallas import tpu_sc as plsc

N = 50000
D = 32
E = 800000
NC = 2
NS = 16
NW = NC * NS
C = 128
NCHUNK = -(-E // (NW * C))
SUP = 28
NSUP = NCHUNK // SUP
E_PAD = NW * NCHUNK * C
N_PAD = 50176
TRASH = N_PAD - N
RPT = N_PAD // NS
R_BLK = 1024
N_BLKS = N_PAD // R_BLK
NLIST = 7

SRC_IS_A = (False, False, True, True, True, False, False)


def _seg_sum_sc(feat_a, feat_b, dst_all, src_all, zeros2d):
    mesh = plsc.VectorSubcoreMesh(
        core_axis_name="c", subcore_axis_name="s", num_cores=NC,
        num_subcores=NS)

    @functools.partial(
        pl.kernel,
        out_type=jax.ShapeDtypeStruct((NLIST, NC, N_PAD, D), jnp.float32),
        mesh=mesh,
        compiler_params=pltpu.CompilerParams(use_tc_tiling_on_sc=False),
        scratch_types=[
            pltpu.VMEM_SHARED((N_PAD, D), jnp.float32),
            pltpu.VMEM((SUP, C), jnp.int32),
            pltpu.VMEM((SUP, C), jnp.int32),
            pltpu.VMEM((C, D), jnp.float32),
            pltpu.VMEM((C, D), jnp.float32),
            pltpu.SemaphoreType.DMA,
            pltpu.SemaphoreType.DMA,
        ],
    )
    def seg_kernel(feat_a_hbm, feat_b_hbm, dst_hbm, src_hbm, z2_hbm,
                   acc_out, acc_sh, dst_buf, src_buf, rows0, rows1,
                   sem0, sem1):
        cid = lax.axis_index("c")
        sid = lax.axis_index("s")
        wid = cid * NS + sid

        for l in range(NLIST):
            feat_hbm = feat_a_hbm if SRC_IS_A[l] else feat_b_hbm

            pltpu.sync_copy(z2_hbm, acc_sh.at[pl.ds(sid * RPT, RPT)])
            plsc.subcore_barrier()

            def gstart(c, buf, sem):
                pltpu.async_copy(feat_hbm.at[src_buf.at[c]], buf, sem)

            def gwait(c, buf, sem):
                pltpu.make_async_copy(feat_hbm.at[src_buf.at[c]], buf,
                                      sem).wait()

            def scat(c, buf):
                pltpu.sync_copy(buf, acc_sh.at[dst_buf.at[c]], add=True)

            def sup_body(sp, carry):
                pltpu.sync_copy(dst_hbm.at[l, wid, sp], dst_buf)
                pltpu.sync_copy(src_hbm.at[l, wid, sp], src_buf)

                gstart(0, rows0, sem0)

                def pair(i, carry):
                    c0 = 2 * i
                    gstart(c0 + 1, rows1, sem1)
                    gwait(c0, rows0, sem0)
                    scat(c0, rows0)
                    gstart(c0 + 2, rows0, sem0)
                    gwait(c0 + 1, rows1, sem1)
                    scat(c0 + 1, rows1)
                    return carry

                lax.fori_loop(0, SUP // 2 - 1, pair, 0)
                c0 = SUP - 2
                gstart(c0 + 1, rows1, sem1)
                gwait(c0, rows0, sem0)
                scat(c0, rows0)
                gwait(c0 + 1, rows1, sem1)
                scat(c0 + 1, rows1)
                return carry

            lax.fori_loop(0, NSUP, sup_body, 0)

            plsc.subcore_barrier()

            pltpu.sync_copy(acc_sh.at[pl.ds(sid * RPT, RPT)],
                            acc_out.at[l, cid, pl.ds(sid * RPT, RPT)])
            plsc.subcore_barrier()

    return seg_kernel(feat_a, feat_b, dst_all, src_all, zeros2d)


DEGW = 16


def _deg_sc(dst_all, ones16, zeros16):
    mesh = plsc.VectorSubcoreMesh(
        core_axis_name="c", subcore_axis_name="s", num_cores=NC,
        num_subcores=NS)

    @functools.partial(
        pl.kernel,
        out_type=jax.ShapeDtypeStruct((NLIST, NC, N_PAD, DEGW), jnp.float32),
        mesh=mesh,
        compiler_params=pltpu.CompilerParams(use_tc_tiling_on_sc=False),
        scratch_types=[
            pltpu.VMEM_SHARED((N_PAD, DEGW), jnp.float32),
            pltpu.VMEM((NCHUNK, C), jnp.int32),
            pltpu.VMEM((C, DEGW), jnp.float32),
        ],
    )
    def deg_kernel(dst_hbm, ones_hbm, z_hbm, deg_out, deg_sh, dst_buf,
                   ones_b):
        cid = lax.axis_index("c")
        sid = lax.axis_index("s")
        wid = cid * NS + sid

        pltpu.sync_copy(ones_hbm, ones_b)

        for l in range(NLIST):
            pltpu.sync_copy(z_hbm, deg_sh.at[pl.ds(sid * RPT, RPT)])
            pltpu.sync_copy(dst_hbm.at[l, wid], dst_buf)
            plsc.subcore_barrier()

            def chunk(c, carry):
                pltpu.sync_copy(ones_b, deg_sh.at[dst_buf.at[c]], add=True)
                return carry

            lax.fori_loop(0, NCHUNK, chunk, 0)

            plsc.subcore_barrier()
            pltpu.sync_copy(deg_sh.at[pl.ds(sid * RPT, RPT)],
                            deg_out.at[l, cid, pl.ds(sid * RPT, RPT)])
            plsc.subcore_barrier()

    return deg_kernel(dst_all.reshape(NLIST, NW, NCHUNK, C), ones16,
                      zeros16)


def _update_tc(feat, accs, degs, slot_map, Ws, bWs, U1_w, U1_b, prelu_a,
               U2_w, U2_b):
    n_agg = len(accs)

    def body(feat_ref, *refs):
        a_refs = refs[:n_agg]
        d_refs = refs[n_agg:2 * n_agg]
        w_refs = refs[2 * n_agg:2 * n_agg + 4]
        bw_refs = refs[2 * n_agg + 4:2 * n_agg + 8]
        u1w_ref, u1b_ref, pa_ref, u2w_ref, u2b_ref, out_ref = \
            refs[2 * n_agg + 8:]

        u1 = u1w_ref[...]
        x = feat_ref[...]
        h = jnp.dot(x, u1[0:D, :], preferred_element_type=jnp.float32)
        h = h + u1b_ref[...]

        means = {}
        for slot in range(4):
            ai = slot_map[slot]
            if ai not in means:
                acc = a_refs[ai][...]
                dg = d_refs[ai][...]
                dsum = jnp.maximum(dg[0, :, 0:1] + dg[1, :, 0:1], 1.0)
                means[ai] = (acc[0] + acc[1]) * (1.0 / dsum)
            s = means[ai]
            u1_blk = u1[D * (slot + 1):D * (slot + 2), :]
            g = jnp.dot(w_refs[slot][...], u1_blk,
                        preferred_element_type=jnp.float32)
            h = h + jnp.dot(s, g, preferred_element_type=jnp.float32)
            h = h + jnp.dot(bw_refs[slot][...], u1_blk,
                            preferred_element_type=jnp.float32)

        a = pa_ref[0, 0]
        h = jnp.where(h >= 0, h, a * h)
        y = jnp.dot(h, u2w_ref[...], preferred_element_type=jnp.float32)
        out_ref[...] = y + u2b_ref[...]

    in_specs = (
        [pl.BlockSpec((R_BLK, D), lambda i: (i, 0))]
        + [pl.BlockSpec((NC, R_BLK, D), lambda i: (0, i, 0))] * n_agg
        + [pl.BlockSpec((NC, R_BLK, DEGW), lambda i: (0, i, 0))] * n_agg
        + [pl.BlockSpec((D, D), lambda i: (0, 0))] * 4
        + [pl.BlockSpec((1, D), lambda i: (0, 0))] * 4
        + [pl.BlockSpec((5 * D, 2 * D), lambda i: (0, 0)),
           pl.BlockSpec((1, 2 * D), lambda i: (0, 0)),
           pl.BlockSpec((1, 1), lambda i: (0, 0)),
           pl.BlockSpec((2 * D, D), lambda i: (0, 0)),
           pl.BlockSpec((1, D), lambda i: (0, 0))]
    )
    return pl.pallas_call(
        body,
        grid=(N_BLKS,),
        in_specs=in_specs,
        out_specs=pl.BlockSpec((R_BLK, D), lambda i: (i, 0)),
        out_shape=jax.ShapeDtypeStruct((N_PAD, D), jnp.float32),
    )(feat, *accs, *degs, *Ws,
      *[b.reshape(1, D) for b in bWs], U1_w, U1_b.reshape(1, 2 * D),
      prelu_a.reshape(1, 1), U2_w, U2_b.reshape(1, D))


def _prep_edges(edges):
    pad = E_PAD - E
    dst = jnp.concatenate(
        [edges[0], N + (jnp.arange(pad, dtype=jnp.int32) % TRASH)])
    src = jnp.concatenate([edges[1], jnp.zeros((pad,), jnp.int32)])
    return (dst.reshape(NW, NSUP, SUP, C), src.reshape(NW, NSUP, SUP, C))


def kernel(feature_a, feature_b, edges_ab_pos, edges_ab_neg, edges_ba_pos,
           edges_ba_neg, edges_aa_pos, edges_aa_neg, edges_bb_pos,
           edges_bb_neg, W0, bW0, W1, bW1, W2, bW2, W3, bW3, W4, bW4, W5,
           bW5, W6, bW6, W7, bW7, U1_w, U1_b, prelu_a, U2_w, U2_b):
    feat_a = jnp.pad(feature_a, ((0, N_PAD - N), (0, 0)))
    feat_b = jnp.pad(feature_b, ((0, N_PAD - N), (0, 0)))

    lists = (edges_ab_pos, edges_ab_neg, edges_aa_pos, edges_aa_neg,
             edges_ba_pos, edges_bb_pos, edges_bb_neg)
    prepped = [_prep_edges(e) for e in lists]
    dst_all = jnp.stack([p[0] for p in prepped])
    src_all = jnp.stack([p[1] for p in prepped])

    zeros2d = jnp.zeros((RPT, D), jnp.float32)

    acc = _seg_sum_sc(feat_a, feat_b, dst_all, src_all, zeros2d)
    deg = _deg_sc(dst_all, jnp.ones((C, DEGW), jnp.float32),
                  jnp.zeros((RPT, DEGW), jnp.float32))

    dg = [deg[l] for l in range(NLIST)]

    new_a = _update_tc(
        feat_a, [acc[0], acc[1], acc[2], acc[3]],
        [dg[0], dg[1], dg[2], dg[3]], (0, 1, 2, 3),
        (W0, W1, W2, W3), (bW0, bW1, bW2, bW3), U1_w, U1_b, prelu_a,
        U2_w, U2_b)
    new_b = _update_tc(
        feat_b, [acc[4], acc[5], acc[6]],
        [dg[4], dg[5], dg[6]], (0, 0, 1, 2),
        (W4, W5, W6, W7), (bW4, bW5, bW6, bW7), U1_w, U1_b, prelu_a,
        U2_w, U2_b)
    return new_a[:N], new_b[:N]


# speedup vs baseline: 14.1275x; 1.2845x over previous
import functools

import jax
import jax.numpy as jnp
from jax import lax
from jax.experimental import pallas as pl
from jax.experimental.pallas import tpu as pltpu
from jax.experimental.pallas import tpu_sc as plsc

N = 50000
D = 32
E = 800000
NC = 2
NS = 16
NW = NC * NS
C = 128
EPW = E // NW
NCHUNK = -(-EPW // C)
SUP = 28
NSUP = NCHUNK // SUP
SUPC = SUP * C
TAIL = EPW - (NSUP - 1) * SUPC
TPAD = SUPC - TAIL
N_PAD = 50176
RPT = N_PAD // NS
R_BLK = 1024
N_BLKS = N_PAD // R_BLK
NLIST = 7

SRC_IS_A = (False, False, True, True, True, False, False)


def _seg_sum_sc(feat_a, feat_b, edge_lists, trash, zeros2d):
    mesh = plsc.VectorSubcoreMesh(
        core_axis_name="c", subcore_axis_name="s", num_cores=NC,
        num_subcores=NS)

    @functools.partial(
        pl.kernel,
        out_type=jax.ShapeDtypeStruct((NLIST, NC, N_PAD, D), jnp.float32),
        mesh=mesh,
        compiler_params=pltpu.CompilerParams(use_tc_tiling_on_sc=False),
        scratch_types=[
            pltpu.VMEM_SHARED((N_PAD, D), jnp.float32),
            pltpu.VMEM((SUPC,), jnp.int32),
            pltpu.VMEM((SUPC,), jnp.int32),
            pltpu.VMEM((C, D), jnp.float32),
            pltpu.VMEM((C, D), jnp.float32),
            pltpu.SemaphoreType.DMA,
            pltpu.SemaphoreType.DMA,
        ],
    )
    def seg_kernel(feat_a_hbm, feat_b_hbm, e0, e1, e2, e3, e4, e5, e6,
                   trash_hbm, z2_hbm, acc_out, acc_sh, dst_buf, src_buf,
                   rows0, rows1, sem0, sem1):
        cid = lax.axis_index("c")
        sid = lax.axis_index("s")
        wid = cid * NS + sid
        base = wid * EPW
        edges = (e0, e1, e2, e3, e4, e5, e6)

        for l in range(NLIST):
            feat_hbm = feat_a_hbm if SRC_IS_A[l] else feat_b_hbm
            e_hbm = edges[l]

            pltpu.sync_copy(z2_hbm, acc_sh.at[pl.ds(sid * RPT, RPT)])
            plsc.subcore_barrier()

            def gstart(c, buf, sem):
                pltpu.async_copy(feat_hbm.at[src_buf.at[pl.ds(c * C, C)]],
                                 buf, sem)

            def gwait(c, buf, sem):
                pltpu.make_async_copy(
                    feat_hbm.at[src_buf.at[pl.ds(c * C, C)]], buf,
                    sem).wait()

            def scat(c, buf):
                pltpu.sync_copy(buf, acc_sh.at[dst_buf.at[pl.ds(c * C, C)]],
                                add=True)

            def run_chunks():
                gstart(0, rows0, sem0)

                def pair(i, carry):
                    c0 = 2 * i
                    gstart(c0 + 1, rows1, sem1)
                    gwait(c0, rows0, sem0)
                    scat(c0, rows0)
                    gstart(c0 + 2, rows0, sem0)
                    gwait(c0 + 1, rows1, sem1)
                    scat(c0 + 1, rows1)
                    return carry

                lax.fori_loop(0, SUP // 2 - 1, pair, 0)
                c0 = SUP - 2
                gstart(c0 + 1, rows1, sem1)
                gwait(c0, rows0, sem0)
                scat(c0, rows0)
                gwait(c0 + 1, rows1, sem1)
                scat(c0 + 1, rows1)

            def sup_body(sp, carry):
                off = base + sp * SUPC
                pltpu.sync_copy(e_hbm.at[0, pl.ds(off, SUPC)], dst_buf)
                pltpu.sync_copy(e_hbm.at[1, pl.ds(off, SUPC)], src_buf)
                run_chunks()
                return carry

            lax.fori_loop(0, NSUP - 1, sup_body, 0)

            off = base + (NSUP - 1) * SUPC
            pltpu.sync_copy(e_hbm.at[0, pl.ds(off, TAIL)],
                            dst_buf.at[pl.ds(0, TAIL)])
            pltpu.sync_copy(e_hbm.at[1, pl.ds(off, TAIL)],
                            src_buf.at[pl.ds(0, TAIL)])
            pltpu.sync_copy(trash_hbm.at[pl.ds(0, TPAD)],
                            dst_buf.at[pl.ds(TAIL, TPAD)])
            pltpu.sync_copy(trash_hbm.at[pl.ds(0, TPAD)],
                            src_buf.at[pl.ds(TAIL, TPAD)])
            run_chunks()

            plsc.subcore_barrier()

            pltpu.sync_copy(acc_sh.at[pl.ds(sid * RPT, RPT)],
                            acc_out.at[l, cid, pl.ds(sid * RPT, RPT)])
            plsc.subcore_barrier()

    return seg_kernel(feat_a, feat_b, *edge_lists, trash, zeros2d)


DEGW = 16


def _deg_sc(edge_lists, trash, ones16, zeros16):
    mesh = plsc.VectorSubcoreMesh(
        core_axis_name="c", subcore_axis_name="s", num_cores=NC,
        num_subcores=NS)

    @functools.partial(
        pl.kernel,
        out_type=jax.ShapeDtypeStruct((NLIST, NC, N_PAD, DEGW), jnp.float32),
        mesh=mesh,
        compiler_params=pltpu.CompilerParams(use_tc_tiling_on_sc=False),
        scratch_types=[
            pltpu.VMEM_SHARED((N_PAD, DEGW), jnp.float32),
            pltpu.VMEM((NCHUNK * C,), jnp.int32),
            pltpu.VMEM((C, DEGW), jnp.float32),
        ],
    )
    def deg_kernel(e0, e1, e2, e3, e4, e5, e6, trash_hbm, ones_hbm, z_hbm,
                   deg_out, deg_sh, dst_buf, ones_b):
        cid = lax.axis_index("c")
        sid = lax.axis_index("s")
        wid = cid * NS + sid
        base = wid * EPW
        edges = (e0, e1, e2, e3, e4, e5, e6)

        pltpu.sync_copy(ones_hbm, ones_b)

        for l in range(NLIST):
            pltpu.sync_copy(z_hbm, deg_sh.at[pl.ds(sid * RPT, RPT)])
            pltpu.sync_copy(edges[l].at[0, pl.ds(base, EPW)],
                            dst_buf.at[pl.ds(0, EPW)])
            pltpu.sync_copy(trash_hbm.at[pl.ds(0, TPAD)],
                            dst_buf.at[pl.ds(EPW, TPAD)])
            plsc.subcore_barrier()

            def chunk(c, carry):
                pltpu.sync_copy(ones_b,
                                deg_sh.at[dst_buf.at[pl.ds(c * C, C)]],
                                add=True)
                return carry

            lax.fori_loop(0, NCHUNK, chunk, 0)

            plsc.subcore_barrier()
            pltpu.sync_copy(deg_sh.at[pl.ds(sid * RPT, RPT)],
                            deg_out.at[l, cid, pl.ds(sid * RPT, RPT)])
            plsc.subcore_barrier()

    return deg_kernel(*edge_lists, trash, ones16, zeros16)


def _update_tc(feat, acc, deg, list_ids, slot_map, Ws, bWs, U1_w, U1_b,
               prelu_a, U2_w, U2_b):
    n_agg = len(list_ids)

    def body(feat_ref, *refs):
        a_refs = refs[:n_agg]
        d_refs = refs[n_agg:2 * n_agg]
        w_refs = refs[2 * n_agg:2 * n_agg + 4]
        bw_refs = refs[2 * n_agg + 4:2 * n_agg + 8]
        u1w_ref, u1b_ref, pa_ref, u2w_ref, u2b_ref, out_ref = \
            refs[2 * n_agg + 8:]

        u1 = u1w_ref[...]
        x = feat_ref[...]
        h = jnp.dot(x, u1[0:D, :], preferred_element_type=jnp.float32)
        h = h + u1b_ref[...]

        means = {}
        for slot in range(4):
            ai = slot_map[slot]
            if ai not in means:
                acc_b = a_refs[ai][0]
                dg = d_refs[ai][0]
                dsum = jnp.maximum(dg[0, :, 0:1] + dg[1, :, 0:1], 1.0)
                means[ai] = (acc_b[0] + acc_b[1]) * (1.0 / dsum)
            s = means[ai]
            u1_blk = u1[D * (slot + 1):D * (slot + 2), :]
            g = jnp.dot(w_refs[slot][...], u1_blk,
                        preferred_element_type=jnp.float32)
            h = h + jnp.dot(s, g, preferred_element_type=jnp.float32)
            h = h + jnp.dot(bw_refs[slot][...], u1_blk,
                            preferred_element_type=jnp.float32)

        a = pa_ref[0, 0]
        h = jnp.where(h >= 0, h, a * h)
        y = jnp.dot(h, u2w_ref[...], preferred_element_type=jnp.float32)
        out_ref[...] = y + u2b_ref[...]

    in_specs = (
        [pl.BlockSpec((R_BLK, D), lambda i: (i, 0))]
        + [pl.BlockSpec((1, NC, R_BLK, D), lambda i, l=l: (l, 0, i, 0))
           for l in list_ids]
        + [pl.BlockSpec((1, NC, R_BLK, DEGW), lambda i, l=l: (l, 0, i, 0))
           for l in list_ids]
        + [pl.BlockSpec((D, D), lambda i: (0, 0))] * 4
        + [pl.BlockSpec((1, D), lambda i: (0, 0))] * 4
        + [pl.BlockSpec((5 * D, 2 * D), lambda i: (0, 0)),
           pl.BlockSpec((1, 2 * D), lambda i: (0, 0)),
           pl.BlockSpec((1, 1), lambda i: (0, 0)),
           pl.BlockSpec((2 * D, D), lambda i: (0, 0)),
           pl.BlockSpec((1, D), lambda i: (0, 0))]
    )
    return pl.pallas_call(
        body,
        grid=(N_BLKS,),
        in_specs=in_specs,
        out_specs=pl.BlockSpec((R_BLK, D), lambda i: (i, 0)),
        out_shape=jax.ShapeDtypeStruct((N_PAD, D), jnp.float32),
    )(feat, *([acc] * n_agg), *([deg] * n_agg), *Ws,
      *[b.reshape(1, D) for b in bWs], U1_w, U1_b.reshape(1, 2 * D),
      prelu_a.reshape(1, 1), U2_w, U2_b.reshape(1, D))


def kernel(feature_a, feature_b, edges_ab_pos, edges_ab_neg, edges_ba_pos,
           edges_ba_neg, edges_aa_pos, edges_aa_neg, edges_bb_pos,
           edges_bb_neg, W0, bW0, W1, bW1, W2, bW2, W3, bW3, W4, bW4, W5,
           bW5, W6, bW6, W7, bW7, U1_w, U1_b, prelu_a, U2_w, U2_b):
    feat_a = jnp.pad(feature_a, ((0, N_PAD - N), (0, 0)))
    feat_b = jnp.pad(feature_b, ((0, N_PAD - N), (0, 0)))

    lists = (edges_ab_pos, edges_ab_neg, edges_aa_pos, edges_aa_neg,
             edges_ba_pos, edges_bb_pos, edges_bb_neg)

    trash = jnp.full((C,), N, jnp.int32)
    zeros2d = jnp.zeros((RPT, D), jnp.float32)

    acc = _seg_sum_sc(feat_a, feat_b, lists, trash, zeros2d)
    deg = _deg_sc(lists, trash, jnp.ones((C, DEGW), jnp.float32),
                  jnp.zeros((RPT, DEGW), jnp.float32))

    new_a = _update_tc(
        feat_a, acc, deg, (0, 1, 2, 3), (0, 1, 2, 3),
        (W0, W1, W2, W3), (bW0, bW1, bW2, bW3), U1_w, U1_b, prelu_a,
        U2_w, U2_b)
    new_b = _update_tc(
        feat_b, acc, deg, (4, 5, 6), (0, 0, 1, 2),
        (W4, W5, W6, W7), (bW4, bW5, bW6, bW7), U1_w, U1_b, prelu_a,
        U2_w, U2_b)
    return new_a[:N], new_b[:N]


# speedup vs baseline: 15.2044x; 1.0762x over previous
import functools

import jax
import jax.numpy as jnp
from jax import lax
from jax.experimental import pallas as pl
from jax.experimental.pallas import tpu as pltpu
from jax.experimental.pallas import tpu_sc as plsc

N = 50000
D = 32
E = 800000
NC = 2
NS = 16
NW = NC * NS
C = 128
EPW = E // NW
NCHUNK = -(-EPW // C)
SUP = 28
NSUP = NCHUNK // SUP
SUPC = SUP * C
TAIL = EPW - (NSUP - 1) * SUPC
TPAD = SUPC - TAIL
N_PAD = 50176
RPT = N_PAD // NS
R_BLK = 1024
N_BLKS = N_PAD // R_BLK
NLIST = 7

SRC_IS_A = (False, False, True, True, True, False, False)


def _seg_sum_sc(feat_a, feat_b, edge_lists, list_ids, trash, zeros2d):
    nl = len(list_ids)
    mesh = plsc.VectorSubcoreMesh(
        core_axis_name="c", subcore_axis_name="s", num_cores=NC,
        num_subcores=NS)

    @functools.partial(
        pl.kernel,
        out_type=jax.ShapeDtypeStruct((nl, NC, N_PAD, D), jnp.float32),
        mesh=mesh,
        compiler_params=pltpu.CompilerParams(use_tc_tiling_on_sc=False),
        scratch_types=[
            pltpu.VMEM_SHARED((N_PAD, D), jnp.float32),
            pltpu.VMEM((SUPC,), jnp.int32),
            pltpu.VMEM((SUPC,), jnp.int32),
            pltpu.VMEM((C, D), jnp.float32),
            pltpu.VMEM((C, D), jnp.float32),
            pltpu.SemaphoreType.DMA,
            pltpu.SemaphoreType.DMA,
        ],
    )
    def seg_kernel(feat_a_hbm, feat_b_hbm, *rest):
        edges = rest[:nl]
        (trash_hbm, z2_hbm, acc_out, acc_sh, dst_buf, src_buf,
         rows0, rows1, sem0, sem1) = rest[nl:]
        cid = lax.axis_index("c")
        sid = lax.axis_index("s")
        wid = cid * NS + sid
        base = wid * EPW

        for l, gl in enumerate(list_ids):
            feat_hbm = feat_a_hbm if SRC_IS_A[gl] else feat_b_hbm
            e_hbm = edges[l]

            pltpu.sync_copy(z2_hbm, acc_sh.at[pl.ds(sid * RPT, RPT)])
            plsc.subcore_barrier()

            def gstart(c, buf, sem):
                pltpu.async_copy(feat_hbm.at[src_buf.at[pl.ds(c * C, C)]],
                                 buf, sem)

            def gwait(c, buf, sem):
                pltpu.make_async_copy(
                    feat_hbm.at[src_buf.at[pl.ds(c * C, C)]], buf,
                    sem).wait()

            def scat(c, buf):
                pltpu.sync_copy(buf, acc_sh.at[dst_buf.at[pl.ds(c * C, C)]],
                                add=True)

            def run_chunks():
                gstart(0, rows0, sem0)

                def pair(i, carry):
                    c0 = 2 * i
                    gstart(c0 + 1, rows1, sem1)
                    gwait(c0, rows0, sem0)
                    scat(c0, rows0)
                    gstart(c0 + 2, rows0, sem0)
                    gwait(c0 + 1, rows1, sem1)
                    scat(c0 + 1, rows1)
                    return carry

                lax.fori_loop(0, SUP // 2 - 1, pair, 0)
                c0 = SUP - 2
                gstart(c0 + 1, rows1, sem1)
                gwait(c0, rows0, sem0)
                scat(c0, rows0)
                gwait(c0 + 1, rows1, sem1)
                scat(c0 + 1, rows1)

            def sup_body(sp, carry):
                off = base + sp * SUPC
                pltpu.sync_copy(e_hbm.at[0, pl.ds(off, SUPC)], dst_buf)
                pltpu.sync_copy(e_hbm.at[1, pl.ds(off, SUPC)], src_buf)
                run_chunks()
                return carry

            lax.fori_loop(0, NSUP - 1, sup_body, 0)

            off = base + (NSUP - 1) * SUPC
            pltpu.sync_copy(e_hbm.at[0, pl.ds(off, TAIL)],
                            dst_buf.at[pl.ds(0, TAIL)])
            pltpu.sync_copy(e_hbm.at[1, pl.ds(off, TAIL)],
                            src_buf.at[pl.ds(0, TAIL)])
            pltpu.sync_copy(trash_hbm.at[pl.ds(0, TPAD)],
                            dst_buf.at[pl.ds(TAIL, TPAD)])
            pltpu.sync_copy(trash_hbm.at[pl.ds(0, TPAD)],
                            src_buf.at[pl.ds(TAIL, TPAD)])
            run_chunks()

            plsc.subcore_barrier()

            pltpu.sync_copy(acc_sh.at[pl.ds(sid * RPT, RPT)],
                            acc_out.at[l, cid, pl.ds(sid * RPT, RPT)])
            plsc.subcore_barrier()

    return seg_kernel(feat_a, feat_b, *edge_lists, trash, zeros2d)


def _update_all(feat_a, feat_b, acc_a, acc_b, deg, Ws, bWs, U1_w, U1_b,
                prelu_a, U2_w, U2_b):
    new_a = _update_tc(
        feat_a, acc_a, deg, (0, 1, 2, 3), (0, 1, 2, 3), (0, 1, 2, 3),
        Ws[:4], bWs[:4], U1_w, U1_b, prelu_a, U2_w, U2_b)
    new_b = _update_tc(
        feat_b, acc_b, deg, (0, 1, 2), (4, 5, 6), (0, 0, 1, 2),
        Ws[4:], bWs[4:], U1_w, U1_b, prelu_a, U2_w, U2_b)
    return new_a, new_b


DEGW = 16


def _deg_sc(edge_lists, trash, ones16, zeros16):
    mesh = plsc.VectorSubcoreMesh(
        core_axis_name="c", subcore_axis_name="s", num_cores=NC,
        num_subcores=NS)

    @functools.partial(
        pl.kernel,
        out_type=jax.ShapeDtypeStruct((NLIST, NC, N_PAD, DEGW), jnp.float32),
        mesh=mesh,
        compiler_params=pltpu.CompilerParams(use_tc_tiling_on_sc=False),
        scratch_types=[
            pltpu.VMEM_SHARED((N_PAD, DEGW), jnp.float32),
            pltpu.VMEM((NCHUNK * C,), jnp.int32),
            pltpu.VMEM((C, DEGW), jnp.float32),
        ],
    )
    def deg_kernel(e0, e1, e2, e3, e4, e5, e6, trash_hbm, ones_hbm, z_hbm,
                   deg_out, deg_sh, dst_buf, ones_b):
        cid = lax.axis_index("c")
        sid = lax.axis_index("s")
        wid = cid * NS + sid
        base = wid * EPW
        edges = (e0, e1, e2, e3, e4, e5, e6)

        pltpu.sync_copy(ones_hbm, ones_b)

        for l in range(NLIST):
            pltpu.sync_copy(z_hbm, deg_sh.at[pl.ds(sid * RPT, RPT)])
            pltpu.sync_copy(edges[l].at[0, pl.ds(base, EPW)],
                            dst_buf.at[pl.ds(0, EPW)])
            pltpu.sync_copy(trash_hbm.at[pl.ds(0, TPAD)],
                            dst_buf.at[pl.ds(EPW, TPAD)])
            plsc.subcore_barrier()

            def chunk(c, carry):
                pltpu.sync_copy(ones_b,
                                deg_sh.at[dst_buf.at[pl.ds(c * C, C)]],
                                add=True)
                return carry

            lax.fori_loop(0, NCHUNK, chunk, 0)

            plsc.subcore_barrier()
            pltpu.sync_copy(deg_sh.at[pl.ds(sid * RPT, RPT)],
                            deg_out.at[l, cid, pl.ds(sid * RPT, RPT)])
            plsc.subcore_barrier()

    return deg_kernel(*edge_lists, trash, ones16, zeros16)


def _update_tc(feat, acc, deg, acc_ids, deg_ids, slot_map, Ws, bWs, U1_w,
               U1_b, prelu_a, U2_w, U2_b):
    n_agg = len(acc_ids)

    def body(feat_ref, *refs):
        a_refs = refs[:n_agg]
        d_refs = refs[n_agg:2 * n_agg]
        w_refs = refs[2 * n_agg:2 * n_agg + 4]
        bw_refs = refs[2 * n_agg + 4:2 * n_agg + 8]
        u1w_ref, u1b_ref, pa_ref, u2w_ref, u2b_ref, out_ref = \
            refs[2 * n_agg + 8:]

        u1 = u1w_ref[...]
        x = feat_ref[...]
        h = jnp.dot(x, u1[0:D, :], preferred_element_type=jnp.float32)
        h = h + u1b_ref[...]

        means = {}
        for slot in range(4):
            ai = slot_map[slot]
            if ai not in means:
                acc_b = a_refs[ai][0]
                dg = d_refs[ai][0]
                dsum = jnp.maximum(dg[0, :, 0:1] + dg[1, :, 0:1], 1.0)
                means[ai] = (acc_b[0] + acc_b[1]) * (1.0 / dsum)
            s = means[ai]
            u1_blk = u1[D * (slot + 1):D * (slot + 2), :]
            g = jnp.dot(w_refs[slot][...], u1_blk,
                        preferred_element_type=jnp.float32)
            h = h + jnp.dot(s, g, preferred_element_type=jnp.float32)
            h = h + jnp.dot(bw_refs[slot][...], u1_blk,
                            preferred_element_type=jnp.float32)

        a = pa_ref[0, 0]
        h = jnp.where(h >= 0, h, a * h)
        y = jnp.dot(h, u2w_ref[...], preferred_element_type=jnp.float32)
        out_ref[...] = y + u2b_ref[...]

    in_specs = (
        [pl.BlockSpec((R_BLK, D), lambda i: (i, 0))]
        + [pl.BlockSpec((1, NC, R_BLK, D), lambda i, l=l: (l, 0, i, 0))
           for l in acc_ids]
        + [pl.BlockSpec((1, NC, R_BLK, DEGW), lambda i, l=l: (l, 0, i, 0))
           for l in deg_ids]
        + [pl.BlockSpec((D, D), lambda i: (0, 0))] * 4
        + [pl.BlockSpec((1, D), lambda i: (0, 0))] * 4
        + [pl.BlockSpec((5 * D, 2 * D), lambda i: (0, 0)),
           pl.BlockSpec((1, 2 * D), lambda i: (0, 0)),
           pl.BlockSpec((1, 1), lambda i: (0, 0)),
           pl.BlockSpec((2 * D, D), lambda i: (0, 0)),
           pl.BlockSpec((1, D), lambda i: (0, 0))]
    )
    return pl.pallas_call(
        body,
        grid=(N_BLKS,),
        in_specs=in_specs,
        out_specs=pl.BlockSpec((R_BLK, D), lambda i: (i, 0)),
        out_shape=jax.ShapeDtypeStruct((N_PAD, D), jnp.float32),
    )(feat, *([acc] * n_agg), *([deg] * n_agg), *Ws,
      *[b.reshape(1, D) for b in bWs], U1_w, U1_b.reshape(1, 2 * D),
      prelu_a.reshape(1, 1), U2_w, U2_b.reshape(1, D))


def kernel(feature_a, feature_b, edges_ab_pos, edges_ab_neg, edges_ba_pos,
           edges_ba_neg, edges_aa_pos, edges_aa_neg, edges_bb_pos,
           edges_bb_neg, W0, bW0, W1, bW1, W2, bW2, W3, bW3, W4, bW4, W5,
           bW5, W6, bW6, W7, bW7, U1_w, U1_b, prelu_a, U2_w, U2_b):
    feat_a = jnp.pad(feature_a, ((0, N_PAD - N), (0, 0)))
    feat_b = jnp.pad(feature_b, ((0, N_PAD - N), (0, 0)))

    lists = (edges_ab_pos, edges_ab_neg, edges_aa_pos, edges_aa_neg,
             edges_ba_pos, edges_bb_pos, edges_bb_neg)

    trash = jnp.full((C,), N, jnp.int32)
    zeros2d = jnp.zeros((RPT, D), jnp.float32)

    deg = _deg_sc(lists, trash, jnp.ones((C, DEGW), jnp.float32),
                  jnp.zeros((RPT, DEGW), jnp.float32))
    acc_a = _seg_sum_sc(feat_a, feat_b, lists[:4], (0, 1, 2, 3), trash,
                        zeros2d)
    acc_b = _seg_sum_sc(feat_a, feat_b, lists[4:], (4, 5, 6), trash,
                        zeros2d)

    new_a, new_b = _update_all(
        feat_a, feat_b, acc_a, acc_b, deg,
        (W0, W1, W2, W3, W4, W5, W6, W7),
        (bW0, bW1, bW2, bW3, bW4, bW5, bW6, bW7),
        U1_w, U1_b, prelu_a, U2_w, U2_b)
    return new_a[:N], new_b[:N]
